# baseline (speedup 1.0000x reference)
import jax
import jax.numpy as jnp
from jax.experimental import pallas as pl
from jax.experimental.pallas import tpu as pltpu

B = 1024
T = 200
OBS = 512
NC = 8
RB = B // NC


def _body(step_ref, obs_hbm, buf_hbm,
          act, rew, logp, val, abuf, rbuf, lbuf, vbuf,
          obs_out, aout, rout, lout, vout,
          *sems):
    step = step_ref[0]
    for c in range(NC):
        pltpu.make_async_copy(
            buf_hbm.at[pl.ds(c * RB, RB)],
            obs_out.at[pl.ds(c * RB, RB)],
            sems[c],
        ).start()
    col = jax.lax.broadcasted_iota(jnp.int32, (B, T), 1)
    mask = col == step
    aout[...] = jnp.where(mask, act[...], abuf[...])
    rout[...] = jnp.where(mask, rew[...], rbuf[...])
    lout[...] = jnp.where(mask, logp[...], lbuf[...])
    colv = jax.lax.broadcasted_iota(jnp.int32, (B, T + 1), 1)
    vout[...] = jnp.where(colv == step, val[...], vbuf[...])
    for c in range(NC):
        pltpu.make_async_copy(
            buf_hbm.at[pl.ds(c * RB, RB)],
            obs_out.at[pl.ds(c * RB, RB)],
            sems[c],
        ).wait()
        pltpu.make_async_copy(
            obs_hbm.at[pl.ds(c * RB, RB)],
            obs_out.at[pl.ds(c * RB, RB), pl.ds(step, 1), :],
            sems[NC + c],
        ).start()
    for c in range(NC):
        pltpu.make_async_copy(
            obs_hbm.at[pl.ds(c * RB, RB)],
            obs_out.at[pl.ds(c * RB, RB), pl.ds(step, 1), :],
            sems[NC + c],
        ).wait()


def kernel(step, obs, action, reward, log_prob, value,
           obs_buf, actions_buf, rewards_buf, log_prob_buf, values_buf):
    step_arr = jnp.asarray(step, dtype=jnp.int32).reshape((1,))
    hbm = pl.BlockSpec(memory_space=pltpu.MemorySpace.HBM)
    vmem = pl.BlockSpec(memory_space=pltpu.MemorySpace.VMEM)

    outs = pl.pallas_call(
        _body,
        grid_spec=pltpu.PrefetchScalarGridSpec(
            num_scalar_prefetch=1,
            in_specs=[hbm, hbm] + [vmem] * 8,
            out_specs=[hbm] + [vmem] * 4,
            scratch_shapes=[pltpu.SemaphoreType.DMA] * (2 * NC),
        ),
        out_shape=(
            jax.ShapeDtypeStruct((B, T, OBS), jnp.float32),
            jax.ShapeDtypeStruct((B, T), jnp.int32),
            jax.ShapeDtypeStruct((B, T), jnp.float32),
            jax.ShapeDtypeStruct((B, T), jnp.float32),
            jax.ShapeDtypeStruct((B, T + 1), jnp.float32),
        ),
    )(step_arr, obs.reshape(B, 1, OBS), obs_buf,
      action.reshape(B, 1), reward.reshape(B, 1),
      log_prob.reshape(B, 1), value.reshape(B, 1),
      actions_buf, rewards_buf, log_prob_buf, values_buf)

    new_obs, new_actions, new_rewards, new_log_prob, new_values = outs
    return (new_obs, new_actions, new_rewards, new_log_prob, new_values)


# speedup vs baseline: 43.5688x; 43.5688x over previous
import jax
import jax.numpy as jnp
from jax.experimental import pallas as pl
from jax.experimental.pallas import tpu as pltpu

B = 1024
T = 200
OBS = 512
BB = 16


def _obs_body(step_ref, obs_blk, buf_blk, out_blk):
    step = step_ref[0]
    tids = jax.lax.broadcasted_iota(jnp.int32, (1, T, 1), 1)
    out_blk[...] = jnp.where(tids == step, obs_blk[...][:, None, :], buf_blk[...])


def _small_body(step_ref, act, rew, logp, val, abuf, rbuf, lbuf, vbuf,
                aout, rout, lout, vout):
    step = step_ref[0]
    col = jax.lax.broadcasted_iota(jnp.int32, (B, T), 1)
    mask = col == step
    aout[...] = jnp.where(mask, act[...], abuf[...])
    rout[...] = jnp.where(mask, rew[...], rbuf[...])
    lout[...] = jnp.where(mask, logp[...], lbuf[...])
    colv = jax.lax.broadcasted_iota(jnp.int32, (B, T + 1), 1)
    vout[...] = jnp.where(colv == step, val[...], vbuf[...])


def kernel(step, obs, action, reward, log_prob, value,
           obs_buf, actions_buf, rewards_buf, log_prob_buf, values_buf):
    step_arr = jnp.asarray(step, dtype=jnp.int32).reshape((1,))

    new_obs = pl.pallas_call(
        _obs_body,
        grid_spec=pltpu.PrefetchScalarGridSpec(
            num_scalar_prefetch=1,
            grid=(B // BB,),
            in_specs=[
                pl.BlockSpec((BB, OBS), lambda i, s: (i, 0)),
                pl.BlockSpec((BB, T, OBS), lambda i, s: (i, 0, 0)),
            ],
            out_specs=pl.BlockSpec((BB, T, OBS), lambda i, s: (i, 0, 0)),
        ),
        out_shape=jax.ShapeDtypeStruct((B, T, OBS), jnp.float32),
        compiler_params=pltpu.CompilerParams(
            dimension_semantics=("arbitrary",),
        ),
    )(step_arr, obs, obs_buf)

    new_actions, new_rewards, new_log_prob, new_values = pl.pallas_call(
        _small_body,
        grid_spec=pltpu.PrefetchScalarGridSpec(num_scalar_prefetch=1),
        out_shape=(
            jax.ShapeDtypeStruct((B, T), jnp.int32),
            jax.ShapeDtypeStruct((B, T), jnp.float32),
            jax.ShapeDtypeStruct((B, T), jnp.float32),
            jax.ShapeDtypeStruct((B, T + 1), jnp.float32),
        ),
    )(step_arr,
      action.reshape(B, 1), reward.reshape(B, 1),
      log_prob.reshape(B, 1), value.reshape(B, 1),
      actions_buf, rewards_buf, log_prob_buf, values_buf)

    return (new_obs, new_actions, new_rewards, new_log_prob, new_values)


# speedup vs baseline: 43.8377x; 1.0062x over previous
import jax
import jax.numpy as jnp
from jax.experimental import pallas as pl
from jax.experimental.pallas import tpu as pltpu

B = 1024
T = 200
OBS = 512
BB = 16


def _body(step_ref, obs_blk, buf_blk, act, rew, logp, val, abuf, rbuf, lbuf, vbuf,
          obs_out, aout, rout, lout, vout):
    step = step_ref[0]
    tids = jax.lax.broadcasted_iota(jnp.int32, (1, T, 1), 1)
    obs_out[...] = jnp.where(tids == step, obs_blk[...][:, None, :], buf_blk[...])
    col = jax.lax.broadcasted_iota(jnp.int32, (BB, T), 1)
    mask = col == step
    aout[...] = jnp.where(mask, act[...], abuf[...])
    rout[...] = jnp.where(mask, rew[...], rbuf[...])
    lout[...] = jnp.where(mask, logp[...], lbuf[...])
    colv = jax.lax.broadcasted_iota(jnp.int32, (BB, T + 1), 1)
    vout[...] = jnp.where(colv == step, val[...], vbuf[...])


def kernel(step, obs, action, reward, log_prob, value,
           obs_buf, actions_buf, rewards_buf, log_prob_buf, values_buf):
    step_arr = jnp.asarray(step, dtype=jnp.int32).reshape((1,))

    def row(i, s):
        return (i, 0)

    outs = pl.pallas_call(
        _body,
        grid_spec=pltpu.PrefetchScalarGridSpec(
            num_scalar_prefetch=1,
            grid=(B // BB,),
            in_specs=[
                pl.BlockSpec((BB, OBS), row),
                pl.BlockSpec((BB, T, OBS), lambda i, s: (i, 0, 0)),
                pl.BlockSpec((BB, 1), row),
                pl.BlockSpec((BB, 1), row),
                pl.BlockSpec((BB, 1), row),
                pl.BlockSpec((BB, 1), row),
                pl.BlockSpec((BB, T), row),
                pl.BlockSpec((BB, T), row),
                pl.BlockSpec((BB, T), row),
                pl.BlockSpec((BB, T + 1), row),
            ],
            out_specs=[
                pl.BlockSpec((BB, T, OBS), lambda i, s: (i, 0, 0)),
                pl.BlockSpec((BB, T), row),
                pl.BlockSpec((BB, T), row),
                pl.BlockSpec((BB, T), row),
                pl.BlockSpec((BB, T + 1), row),
            ],
        ),
        out_shape=(
            jax.ShapeDtypeStruct((B, T, OBS), jnp.float32),
            jax.ShapeDtypeStruct((B, T), jnp.int32),
            jax.ShapeDtypeStruct((B, T), jnp.float32),
            jax.ShapeDtypeStruct((B, T), jnp.float32),
            jax.ShapeDtypeStruct((B, T + 1), jnp.float32),
        ),
        compiler_params=pltpu.CompilerParams(
            dimension_semantics=("arbitrary",),
        ),
    )(step_arr, obs, obs_buf,
      action.reshape(B, 1), reward.reshape(B, 1),
      log_prob.reshape(B, 1), value.reshape(B, 1),
      actions_buf, rewards_buf, log_prob_buf, values_buf)

    new_obs, new_actions, new_rewards, new_log_prob, new_values = outs
    return (new_obs, new_actions, new_rewards, new_log_prob, new_values)


# speedup vs baseline: 44.0120x; 1.0040x over previous
import jax
import jax.numpy as jnp
from jax.experimental import pallas as pl
from jax.experimental.pallas import tpu as pltpu

B = 1024
T = 200
OBS = 512
BB = 32


def _body(step_ref, obs_blk, buf_blk, act, rew, logp, val, abuf, rbuf, lbuf, vbuf,
          obs_out, aout, rout, lout, vout):
    step = step_ref[0]
    tids = jax.lax.broadcasted_iota(jnp.int32, (1, T, 1), 1)
    obs_out[...] = jnp.where(tids == step, obs_blk[...][:, None, :], buf_blk[...])
    col = jax.lax.broadcasted_iota(jnp.int32, (BB, T), 1)
    mask = col == step
    aout[...] = jnp.where(mask, act[...], abuf[...])
    rout[...] = jnp.where(mask, rew[...], rbuf[...])
    lout[...] = jnp.where(mask, logp[...], lbuf[...])
    colv = jax.lax.broadcasted_iota(jnp.int32, (BB, T + 1), 1)
    vout[...] = jnp.where(colv == step, val[...], vbuf[...])


def kernel(step, obs, action, reward, log_prob, value,
           obs_buf, actions_buf, rewards_buf, log_prob_buf, values_buf):
    step_arr = jnp.asarray(step, dtype=jnp.int32).reshape((1,))

    def row(i, s):
        return (i, 0)

    outs = pl.pallas_call(
        _body,
        grid_spec=pltpu.PrefetchScalarGridSpec(
            num_scalar_prefetch=1,
            grid=(B // BB,),
            in_specs=[
                pl.BlockSpec((BB, OBS), row),
                pl.BlockSpec((BB, T, OBS), lambda i, s: (i, 0, 0)),
                pl.BlockSpec((BB, 1), row),
                pl.BlockSpec((BB, 1), row),
                pl.BlockSpec((BB, 1), row),
                pl.BlockSpec((BB, 1), row),
                pl.BlockSpec((BB, T), row),
                pl.BlockSpec((BB, T), row),
                pl.BlockSpec((BB, T), row),
                pl.BlockSpec((BB, T + 1), row),
            ],
            out_specs=[
                pl.BlockSpec((BB, T, OBS), lambda i, s: (i, 0, 0)),
                pl.BlockSpec((BB, T), row),
                pl.BlockSpec((BB, T), row),
                pl.BlockSpec((BB, T), row),
                pl.BlockSpec((BB, T + 1), row),
            ],
        ),
        out_shape=(
            jax.ShapeDtypeStruct((B, T, OBS), jnp.float32),
            jax.ShapeDtypeStruct((B, T), jnp.int32),
            jax.ShapeDtypeStruct((B, T), jnp.float32),
            jax.ShapeDtypeStruct((B, T), jnp.float32),
            jax.ShapeDtypeStruct((B, T + 1), jnp.float32),
        ),
        compiler_params=pltpu.CompilerParams(
            dimension_semantics=("arbitrary",),
        ),
    )(step_arr, obs, obs_buf,
      action.reshape(B, 1), reward.reshape(B, 1),
      log_prob.reshape(B, 1), value.reshape(B, 1),
      actions_buf, rewards_buf, log_prob_buf, values_buf)

    new_obs, new_actions, new_rewards, new_log_prob, new_values = outs
    return (new_obs, new_actions, new_rewards, new_log_prob, new_values)
